# Initial kernel scaffold; baseline (speedup 1.0000x reference)
import jax
import jax.numpy as jnp
from jax.experimental import pallas as pl


def kernel(z, W):
    raise NotImplementedError("write your pallas kernel here")


# speedup vs baseline: 1.2389x; 1.2389x over previous
import functools

import jax
import jax.numpy as jnp
from jax import lax
from jax.experimental import pallas as pl
from jax.experimental.pallas import tpu as pltpu
from jax.experimental.pallas import tpu_sc as plsc

N = 16384
K = 8192
D = 256
BETA = 0.5
EPS = 1e-12

BM = 1024
BK = 1024
NRB = N // BM
NKB = K // BK

SC_CHUNK = 256


def _wnorm_body(w_ref, wn_ref):
    w = w_ref[...]
    s = jnp.sum(w * w, axis=1, keepdims=True)
    wn_ref[...] = w / jnp.maximum(jnp.sqrt(s), EPS)


def _simil_body(z_ref, wn_ref, idx_ref, loss_ref, zn_s, max_s, arg_s, acc_s):
    i = pl.program_id(0)
    k = pl.program_id(1)

    @pl.when(k == 0)
    def _init():
        z = z_ref[...]
        s = jnp.sum(z * z, axis=1, keepdims=True)
        zn_s[...] = z / jnp.maximum(jnp.sqrt(s), EPS)
        max_s[...] = jnp.full((BM, 1), -3.0, jnp.float32)
        arg_s[...] = jnp.zeros((BM, 1), jnp.int32)

    s = lax.dot_general(
        zn_s[...], wn_ref[...],
        dimension_numbers=(((1,), (1,)), ((), ())),
        preferred_element_type=jnp.float32,
    )
    lm = jnp.max(s, axis=1, keepdims=True)
    cols = lax.broadcasted_iota(jnp.int32, (BM, BK), 1)
    la = jnp.min(jnp.where(s == lm, cols, BK), axis=1, keepdims=True) + k * BK
    better = lm > max_s[...]
    arg_s[...] = jnp.where(better, la, arg_s[...])
    max_s[...] = jnp.where(better, lm, max_s[...])

    @pl.when(k == NKB - 1)
    def _fin():
        idx_ref[...] = arg_s[...]
        part = jnp.sum(2.0 - 2.0 * max_s[...])
        prev = jnp.where(i == 0, 0.0, acc_s[0])
        acc_s[0] = prev + part

        @pl.when(i == NRB - 1)
        def _loss():
            loss_ref[0, 0] = (BETA + 1.0) * acc_s[0] / (N * D)


def _wnorm(W):
    return pl.pallas_call(
        _wnorm_body,
        grid=(K // 1024,),
        in_specs=[pl.BlockSpec((1024, D), lambda i: (i, 0))],
        out_specs=pl.BlockSpec((1024, D), lambda i: (i, 0)),
        out_shape=jax.ShapeDtypeStruct((K, D), jnp.float32),
    )(W)


def _simil(z, Wn):
    return pl.pallas_call(
        _simil_body,
        grid=(NRB, NKB),
        in_specs=[
            pl.BlockSpec((BM, D), lambda i, k: (i, 0)),
            pl.BlockSpec((BK, D), lambda i, k: (k, 0)),
        ],
        out_specs=[
            pl.BlockSpec((BM, 1), lambda i, k: (i, 0)),
            pl.BlockSpec((1, 1), lambda i, k: (0, 0), memory_space=pltpu.SMEM),
        ],
        out_shape=[
            jax.ShapeDtypeStruct((N, 1), jnp.int32),
            jax.ShapeDtypeStruct((1, 1), jnp.float32),
        ],
        scratch_shapes=[
            pltpu.VMEM((BM, D), jnp.float32),
            pltpu.VMEM((BM, 1), jnp.float32),
            pltpu.VMEM((BM, 1), jnp.int32),
            pltpu.SMEM((1,), jnp.float32),
        ],
    )(z, Wn)


def _gather(Wn, idx):
    info = plsc.get_sparse_core_info()
    nw = info.num_cores * info.num_subcores
    b_per_w = N // nw
    nchunk = b_per_w // SC_CHUNK
    mesh = plsc.VectorSubcoreMesh(core_axis_name="c", subcore_axis_name="s")

    @functools.partial(
        pl.kernel,
        mesh=mesh,
        out_type=jax.ShapeDtypeStruct((N, D), jnp.float32),
        scratch_types=[
            pltpu.VMEM((SC_CHUNK,), jnp.int32),
            pltpu.VMEM((SC_CHUNK, D), jnp.float32),
            pltpu.SemaphoreType.DMA,
        ],
    )
    def k(table_hbm, idx_hbm, out_hbm, idx_v, rows_v, sem):
        wid = lax.axis_index("s") * info.num_cores + lax.axis_index("c")
        for c in range(nchunk):
            base = wid * b_per_w + c * SC_CHUNK
            pltpu.sync_copy(idx_hbm.at[pl.ds(base, SC_CHUNK)], idx_v)
            pltpu.async_copy(table_hbm.at[idx_v], rows_v, sem).wait()
            pltpu.sync_copy(rows_v, out_hbm.at[pl.ds(base, SC_CHUNK)])

    return k(Wn, idx)


def kernel(z, W):
    Wn = _wnorm(W)
    idx2d, loss2d = _simil(z, Wn)
    idx = idx2d.reshape(N)
    zq = _gather(Wn, idx)
    return (zq, idx, loss2d.reshape(()))


# speedup vs baseline: 1.4713x; 1.1877x over previous
import functools

import jax
import jax.numpy as jnp
from jax import lax
from jax.experimental import pallas as pl
from jax.experimental.pallas import tpu as pltpu
from jax.experimental.pallas import tpu_sc as plsc

N = 16384
H = N // 2
K = 8192
D = 256
BETA = 0.5
EPS = 1e-12

BM = 1024
NRB = H // BM

SC_CHUNK = 256


def _simil_body(norm_w, z_ref, w_ref, idx_ref, sum_ref, *rest):
    if norm_w:
        wn_ref, acc_s = rest
    else:
        (acc_s,) = rest
        wn_ref = w_ref
    i = pl.program_id(0)

    if norm_w:
        @pl.when(i == 0)
        def _wn():
            w = w_ref[...]
            s2 = jnp.sum(w * w, axis=1, keepdims=True)
            wn_ref[...] = w / jnp.maximum(jnp.sqrt(s2), EPS)

    z = z_ref[...]
    sz = jnp.sum(z * z, axis=1, keepdims=True)
    zn = z / jnp.maximum(jnp.sqrt(sz), EPS)
    s = lax.dot_general(
        zn, wn_ref[...],
        dimension_numbers=(((1,), (1,)), ((), ())),
        preferred_element_type=jnp.float32,
    )
    lm = jnp.max(s, axis=1, keepdims=True)
    cols = lax.broadcasted_iota(jnp.int32, (BM, K), 1)
    la = jnp.min(jnp.where(s == lm, cols, K), axis=1, keepdims=True)
    idx_ref[...] = la
    part = jnp.sum(2.0 - 2.0 * lm)
    prev = jnp.where(i == 0, 0.0, acc_s[0])
    acc_s[0] = prev + part

    @pl.when(i == NRB - 1)
    def _sum():
        sum_ref[0, 0] = acc_s[0]


def _simil_first(z0, W):
    return pl.pallas_call(
        functools.partial(_simil_body, True),
        grid=(NRB,),
        in_specs=[
            pl.BlockSpec((BM, D), lambda i: (i, 0)),
            pl.BlockSpec((K, D), lambda i: (0, 0)),
        ],
        out_specs=[
            pl.BlockSpec((BM, 1), lambda i: (i, 0)),
            pl.BlockSpec((1, 1), lambda i: (0, 0), memory_space=pltpu.SMEM),
            pl.BlockSpec((K, D), lambda i: (0, 0)),
        ],
        out_shape=[
            jax.ShapeDtypeStruct((H, 1), jnp.int32),
            jax.ShapeDtypeStruct((1, 1), jnp.float32),
            jax.ShapeDtypeStruct((K, D), jnp.float32),
        ],
        scratch_shapes=[pltpu.SMEM((1,), jnp.float32)],
    )(z0, W)


def _simil_second(z1, Wn):
    return pl.pallas_call(
        functools.partial(_simil_body, False),
        grid=(NRB,),
        in_specs=[
            pl.BlockSpec((BM, D), lambda i: (i, 0)),
            pl.BlockSpec((K, D), lambda i: (0, 0)),
        ],
        out_specs=[
            pl.BlockSpec((BM, 1), lambda i: (i, 0)),
            pl.BlockSpec((1, 1), lambda i: (0, 0), memory_space=pltpu.SMEM),
        ],
        out_shape=[
            jax.ShapeDtypeStruct((H, 1), jnp.int32),
            jax.ShapeDtypeStruct((1, 1), jnp.float32),
        ],
        scratch_shapes=[pltpu.SMEM((1,), jnp.float32)],
    )(z1, Wn)


def _gather(Wn, idx):
    info = plsc.get_sparse_core_info()
    nw = info.num_cores * info.num_subcores
    b_per_w = H // nw
    nchunk = b_per_w // SC_CHUNK
    mesh = plsc.VectorSubcoreMesh(core_axis_name="c", subcore_axis_name="s")

    @functools.partial(
        pl.kernel,
        mesh=mesh,
        out_type=jax.ShapeDtypeStruct((H, D), jnp.float32),
        scratch_types=[
            pltpu.VMEM((SC_CHUNK,), jnp.int32),
            pltpu.VMEM((SC_CHUNK, D), jnp.float32),
            pltpu.SemaphoreType.DMA,
        ],
    )
    def k(table_hbm, idx_hbm, out_hbm, idx_v, rows_v, sem):
        wid = lax.axis_index("s") * info.num_cores + lax.axis_index("c")
        for c in range(nchunk):
            base = wid * b_per_w + c * SC_CHUNK
            pltpu.sync_copy(idx_hbm.at[pl.ds(base, SC_CHUNK)], idx_v)
            pltpu.async_copy(table_hbm.at[idx_v], rows_v, sem).wait()
            pltpu.sync_copy(rows_v, out_hbm.at[pl.ds(base, SC_CHUNK)])

    return k(Wn, idx)


def kernel(z, W):
    idx0_2d, p0, Wn = _simil_first(z[:H], W)
    idx0 = idx0_2d.reshape(H)
    zq0 = _gather(Wn, idx0)
    idx1_2d, p1 = _simil_second(z[H:], Wn)
    idx1 = idx1_2d.reshape(H)
    zq1 = _gather(Wn, idx1)
    zq = jnp.concatenate([zq0, zq1], axis=0)
    idx = jnp.concatenate([idx0, idx1], axis=0)
    loss = ((BETA + 1.0) * (p0 + p1) / (N * D)).reshape(())
    return (zq, idx, loss)


# speedup vs baseline: 1.6816x; 1.1429x over previous
import functools

import jax
import jax.numpy as jnp
from jax import lax
from jax.experimental import pallas as pl
from jax.experimental.pallas import tpu as pltpu
from jax.experimental.pallas import tpu_sc as plsc

N = 16384
K = 8192
D = 256
BETA = 0.5
EPS = 1e-12

BM = 1024
BK = 8192
NRB = N // BM
NKB = K // BK

SC_CHUNK = 256


def _simil_body(z_ref, w_ref, idx_ref, loss_ref, wn_ref, acc_s):
    i = pl.program_id(0)

    @pl.when(i == 0)
    def _wn():
        w = w_ref[...]
        s2 = jnp.sum(w * w, axis=1, keepdims=True)
        wn_ref[...] = w / jnp.maximum(jnp.sqrt(s2), EPS)

    z = z_ref[...]
    sz = jnp.sum(z * z, axis=1, keepdims=True)
    zn = z / jnp.maximum(jnp.sqrt(sz), EPS)
    s = lax.dot_general(
        zn, wn_ref[...],
        dimension_numbers=(((1,), (1,)), ((), ())),
        preferred_element_type=jnp.float32,
    )
    lm = jnp.max(s, axis=1, keepdims=True)
    cols = lax.broadcasted_iota(jnp.int32, (BM, K), 1)
    la = jnp.min(jnp.where(s == lm, cols, K), axis=1, keepdims=True)
    idx_ref[...] = la
    part = jnp.sum(2.0 - 2.0 * lm)
    prev = jnp.where(i == 0, 0.0, acc_s[0])
    acc_s[0] = prev + part

    @pl.when(i == NRB - 1)
    def _loss():
        loss_ref[0, 0] = (BETA + 1.0) * acc_s[0] / (N * D)


def _simil(z, W):
    return pl.pallas_call(
        _simil_body,
        grid=(NRB,),
        in_specs=[
            pl.BlockSpec((BM, D), lambda i: (i, 0)),
            pl.BlockSpec((K, D), lambda i: (0, 0)),
        ],
        out_specs=[
            pl.BlockSpec((BM, 1), lambda i: (i, 0)),
            pl.BlockSpec((1, 1), lambda i: (0, 0), memory_space=pltpu.SMEM),
            pl.BlockSpec((K, D), lambda i: (0, 0)),
        ],
        out_shape=[
            jax.ShapeDtypeStruct((N, 1), jnp.int32),
            jax.ShapeDtypeStruct((1, 1), jnp.float32),
            jax.ShapeDtypeStruct((K, D), jnp.float32),
        ],
        scratch_shapes=[
            pltpu.SMEM((1,), jnp.float32),
        ],
    )(z, W)


def _gather(Wn, idx):
    info = plsc.get_sparse_core_info()
    nw = info.num_cores * info.num_subcores
    b_per_w = N // nw
    nchunk = b_per_w // SC_CHUNK
    mesh = plsc.VectorSubcoreMesh(core_axis_name="c", subcore_axis_name="s")

    @functools.partial(
        pl.kernel,
        mesh=mesh,
        out_type=jax.ShapeDtypeStruct((N, D), jnp.float32),
        scratch_types=[
            pltpu.VMEM((SC_CHUNK,), jnp.int32),
            pltpu.VMEM((SC_CHUNK, D), jnp.float32),
            pltpu.SemaphoreType.DMA,
        ],
    )
    def k(table_hbm, idx_hbm, out_hbm, idx_v, rows_v, sem):
        wid = lax.axis_index("s") * info.num_cores + lax.axis_index("c")
        for c in range(nchunk):
            base = wid * b_per_w + c * SC_CHUNK
            pltpu.sync_copy(idx_hbm.at[pl.ds(base, SC_CHUNK)], idx_v)
            pltpu.async_copy(table_hbm.at[idx_v], rows_v, sem).wait()
            pltpu.sync_copy(rows_v, out_hbm.at[pl.ds(base, SC_CHUNK)])

    return k(Wn, idx)


def kernel(z, W):
    idx2d, loss2d, Wn = _simil(z, W)
    idx = idx2d.reshape(N)
    zq = _gather(Wn, idx)
    return (zq, idx, loss2d.reshape(()))


# speedup vs baseline: 2.7720x; 1.6485x over previous
import functools

import jax
import jax.numpy as jnp
from jax import lax
from jax.experimental import pallas as pl
from jax.experimental.pallas import tpu as pltpu
from jax.experimental.pallas import tpu_sc as plsc

N = 16384
K = 8192
D = 256
BETA = 0.5
EPS = 1e-12

BM = 1024
BK = 8192
NRB = N // BM
NKB = K // BK

SC_CHUNK = 256


def _simil_body(z_ref, w_ref, idx_ref, loss_ref, wn_ref, st_ref, acc_s):
    i = pl.program_id(0)

    @pl.when(i == 0)
    def _wn():
        w = w_ref[...]
        s2 = jnp.sum(w * w, axis=1, keepdims=True)
        wn_ref[...] = w / jnp.maximum(jnp.sqrt(s2), EPS)

    z = z_ref[...]
    sz = jnp.sum(z * z, axis=1, keepdims=True)
    zn = z / jnp.maximum(jnp.sqrt(sz), EPS)
    st_ref[...] = lax.dot_general(
        wn_ref[...], zn,
        dimension_numbers=(((1,), (1,)), ((), ())),
        preferred_element_type=jnp.float32,
    )
    av = jnp.full((8, BM), -3.0, jnp.float32)
    ac = jnp.zeros((8, BM), jnp.int32)
    for v in range(K // 8):
        sv = st_ref[v * 8:(v + 1) * 8, :]
        b = sv > av
        ac = jnp.where(b, v, ac)
        av = jnp.where(b, sv, av)
    rows = lax.broadcasted_iota(jnp.int32, (8, BM), 0)
    g = ac * 8 + rows
    m = jnp.max(av, axis=0, keepdims=True)
    la = jnp.min(jnp.where(av == m, g, K), axis=0, keepdims=True)
    idx_ref[...] = la.reshape(1, 1, BM)
    part = jnp.sum(2.0 - 2.0 * m)
    prev = jnp.where(i == 0, 0.0, acc_s[0])
    acc_s[0] = prev + part

    @pl.when(i == NRB - 1)
    def _loss():
        loss_ref[0, 0] = (BETA + 1.0) * acc_s[0] / (N * D)


def _simil(z, W):
    return pl.pallas_call(
        _simil_body,
        grid=(NRB,),
        in_specs=[
            pl.BlockSpec((BM, D), lambda i: (i, 0)),
            pl.BlockSpec((K, D), lambda i: (0, 0)),
        ],
        out_specs=[
            pl.BlockSpec((1, 1, BM), lambda i: (i, 0, 0)),
            pl.BlockSpec((1, 1), lambda i: (0, 0), memory_space=pltpu.SMEM),
            pl.BlockSpec((K, D), lambda i: (0, 0)),
        ],
        out_shape=[
            jax.ShapeDtypeStruct((NRB, 1, BM), jnp.int32),
            jax.ShapeDtypeStruct((1, 1), jnp.float32),
            jax.ShapeDtypeStruct((K, D), jnp.float32),
        ],
        scratch_shapes=[
            pltpu.VMEM((K, BM), jnp.float32),
            pltpu.SMEM((1,), jnp.float32),
        ],
    )(z, W)


def _gather(Wn, idx):
    info = plsc.get_sparse_core_info()
    nw = info.num_cores * info.num_subcores
    b_per_w = N // nw
    nchunk = b_per_w // SC_CHUNK
    mesh = plsc.VectorSubcoreMesh(core_axis_name="c", subcore_axis_name="s")

    @functools.partial(
        pl.kernel,
        mesh=mesh,
        out_type=jax.ShapeDtypeStruct((N, D), jnp.float32),
        scratch_types=[
            pltpu.VMEM((SC_CHUNK,), jnp.int32),
            pltpu.VMEM((SC_CHUNK, D), jnp.float32),
            pltpu.SemaphoreType.DMA,
        ],
    )
    def k(table_hbm, idx_hbm, out_hbm, idx_v, rows_v, sem):
        wid = lax.axis_index("s") * info.num_cores + lax.axis_index("c")
        for c in range(nchunk):
            base = wid * b_per_w + c * SC_CHUNK
            pltpu.sync_copy(idx_hbm.at[pl.ds(base, SC_CHUNK)], idx_v)
            pltpu.async_copy(table_hbm.at[idx_v], rows_v, sem).wait()
            pltpu.sync_copy(rows_v, out_hbm.at[pl.ds(base, SC_CHUNK)])

    return k(Wn, idx)


def kernel(z, W):
    idx3d, loss2d, Wn = _simil(z, W)
    idx = idx3d.reshape(N)
    zq = _gather(Wn, idx)
    return (zq, idx, loss2d.reshape(()))
